# baseline (speedup 1.0000x reference)
import jax
import jax.numpy as jnp
from jax.experimental import pallas as pl
from jax.experimental.pallas import tpu as pltpu

_PERIODS = 12
_RESOLUTION_S = 3600.0
_SLOTS = _PERIODS + 1


def _body(s_ref, past, latest, upd, dmax, dzero, s0, s1, s2, s3):
    shift = s_ref[0]

    @pl.when(shift > 0)
    def _():
        a = pltpu.make_async_copy(past.at[pl.ds(1, _PERIODS)], upd.at[pl.ds(0, _PERIODS)], s0)
        b = pltpu.make_async_copy(latest, upd.at[_PERIODS], s1)
        c = pltpu.make_async_copy(past.at[1], dmax, s2)
        d = pltpu.make_async_copy(latest, dzero, s3)
        a.start(); b.start(); c.start(); d.start()
        a.wait(); b.wait(); c.wait(); d.wait()

    @pl.when(shift == 0)
    def _():
        a = pltpu.make_async_copy(past, upd, s0)
        c = pltpu.make_async_copy(past.at[0], dmax, s2)
        d = pltpu.make_async_copy(past.at[_PERIODS], dzero, s3)
        a.start(); c.start(); d.start()
        a.wait(); c.wait(); d.wait()


def kernel(past, latest, dt_mod_freq, timedelta_seconds):
    dt = dt_mod_freq[0] + jnp.float32(timedelta_seconds)
    is_update_step = dt >= _RESOLUTION_S
    new_dt = jnp.where(is_update_step, dt - _RESOLUTION_S, dt)
    shift = is_update_step.astype(jnp.int32).reshape((1,))
    field = jax.ShapeDtypeStruct(latest.shape, latest.dtype)
    updated_past, diag_max, diag_zero = pl.pallas_call(
        _body,
        in_specs=[
            pl.BlockSpec(memory_space=pltpu.MemorySpace.SMEM),
            pl.BlockSpec(memory_space=pl.ANY),
            pl.BlockSpec(memory_space=pl.ANY),
        ],
        out_specs=[
            pl.BlockSpec(memory_space=pl.ANY),
            pl.BlockSpec(memory_space=pl.ANY),
            pl.BlockSpec(memory_space=pl.ANY),
        ],
        out_shape=[jax.ShapeDtypeStruct(past.shape, past.dtype), field, field],
        scratch_shapes=[pltpu.SemaphoreType.DMA] * 4,
    )(shift, past, latest)
    return updated_past, diag_max, diag_zero, new_dt


# speedup vs baseline: 30.3838x; 30.3838x over previous
import functools

import jax
import jax.numpy as jnp
from jax.experimental import pallas as pl
from jax.experimental.pallas import tpu as pltpu

_PERIODS = 12
_RESOLUTION_S = 3600.0
_SLOTS = _PERIODS + 1

_CH = 2


def _body(s_ref, past_ref, latest_ref, upd_ref, dmax_ref, dzero_ref):
    i = pl.program_id(0)
    j = pl.program_id(1)
    shift = s_ref[0]
    use_latest = jnp.logical_and(shift > 0, i == _PERIODS)
    val = jnp.where(use_latest, latest_ref[pl.ds(j * _CH, _CH)], past_ref[0])
    upd_ref[0] = val

    @pl.when(i == 0)
    def _():
        dmax_ref[pl.ds(j * _CH, _CH)] = val

    @pl.when(i == _PERIODS)
    def _():
        dzero_ref[pl.ds(j * _CH, _CH)] = val


@functools.partial(jax.jit, static_argnames=())
def _run(past, latest, shift):
    f0, f1, f2 = past.shape[1:]
    jgrid = f0 // _CH
    grid = (_SLOTS, jgrid)
    field = jax.ShapeDtypeStruct((f0, f1, f2), past.dtype)
    return pl.pallas_call(
        _body,
        grid_spec=pltpu.PrefetchScalarGridSpec(
            num_scalar_prefetch=1,
            grid=grid,
            in_specs=[
                pl.BlockSpec(
                    (1, _CH, f1, f2),
                    lambda i, j, s: (jnp.minimum(i + s[0], _SLOTS - 1), j, 0, 0),
                ),
                pl.BlockSpec((f0, f1, f2), lambda i, j, s: (0, 0, 0)),
            ],
            out_specs=[
                pl.BlockSpec((1, _CH, f1, f2), lambda i, j, s: (i, j, 0, 0)),
                pl.BlockSpec((f0, f1, f2), lambda i, j, s: (0, 0, 0)),
                pl.BlockSpec((f0, f1, f2), lambda i, j, s: (0, 0, 0)),
            ],
        ),
        out_shape=[
            jax.ShapeDtypeStruct(past.shape, past.dtype),
            field,
            field,
        ],
    )(shift, past, latest)


def kernel(past, latest, dt_mod_freq, timedelta_seconds):
    dt = dt_mod_freq[0] + jnp.float32(timedelta_seconds)
    is_update_step = dt >= _RESOLUTION_S
    new_dt = jnp.where(is_update_step, dt - _RESOLUTION_S, dt)
    shift = is_update_step.astype(jnp.int32).reshape((1,))
    updated_past, diag_max, diag_zero = _run(past, latest, shift)
    return updated_past, diag_max, diag_zero, new_dt


# speedup vs baseline: 45.8654x; 1.5095x over previous
import functools

import jax
import jax.numpy as jnp
from jax.experimental import pallas as pl
from jax.experimental.pallas import tpu as pltpu

_PERIODS = 12
_RESOLUTION_S = 3600.0
_SLOTS = _PERIODS + 1

_CH = 8


def _body(s_ref, past_ref, latest_ref, upd_ref, dmax_ref, dzero_ref):
    i = pl.program_id(0)
    j = pl.program_id(1)
    shift = s_ref[0]
    use_latest = jnp.logical_and(shift > 0, i == _PERIODS)
    val = jnp.where(use_latest, latest_ref[pl.ds(j * _CH, _CH)], past_ref[0])
    upd_ref[0] = val

    @pl.when(i == 0)
    def _():
        dmax_ref[pl.ds(j * _CH, _CH)] = val

    @pl.when(i == _PERIODS)
    def _():
        dzero_ref[pl.ds(j * _CH, _CH)] = val


@functools.partial(jax.jit, static_argnames=())
def _run(past, latest, shift):
    f0, f1, f2 = past.shape[1:]
    jgrid = f0 // _CH
    grid = (_SLOTS, jgrid)
    field = jax.ShapeDtypeStruct((f0, f1, f2), past.dtype)
    return pl.pallas_call(
        _body,
        grid_spec=pltpu.PrefetchScalarGridSpec(
            num_scalar_prefetch=1,
            grid=grid,
            in_specs=[
                pl.BlockSpec(
                    (1, _CH, f1, f2),
                    lambda i, j, s: (jnp.minimum(i + s[0], _SLOTS - 1), j, 0, 0),
                ),
                pl.BlockSpec((f0, f1, f2), lambda i, j, s: (0, 0, 0)),
            ],
            out_specs=[
                pl.BlockSpec((1, _CH, f1, f2), lambda i, j, s: (i, j, 0, 0)),
                pl.BlockSpec((f0, f1, f2), lambda i, j, s: (0, 0, 0)),
                pl.BlockSpec((f0, f1, f2), lambda i, j, s: (0, 0, 0)),
            ],
        ),
        out_shape=[
            jax.ShapeDtypeStruct(past.shape, past.dtype),
            field,
            field,
        ],
    )(shift, past, latest)


def kernel(past, latest, dt_mod_freq, timedelta_seconds):
    dt = dt_mod_freq[0] + jnp.float32(timedelta_seconds)
    is_update_step = dt >= _RESOLUTION_S
    new_dt = jnp.where(is_update_step, dt - _RESOLUTION_S, dt)
    shift = is_update_step.astype(jnp.int32).reshape((1,))
    updated_past, diag_max, diag_zero = _run(past, latest, shift)
    return updated_past, diag_max, diag_zero, new_dt


# speedup vs baseline: 46.4787x; 1.0134x over previous
import functools

import jax
import jax.numpy as jnp
from jax.experimental import pallas as pl
from jax.experimental.pallas import tpu as pltpu

_PERIODS = 12
_RESOLUTION_S = 3600.0
_SLOTS = _PERIODS + 1

_CH = 8


def _body(s_ref, past_ref, latest_ref, upd_ref, dmax_ref, dzero_ref):
    i = pl.program_id(0)
    j = pl.program_id(1)
    shift = s_ref[0]

    @pl.when(i < _PERIODS)
    def _():
        upd_ref[0] = past_ref[0]

    @pl.when(i == 0)
    def _():
        dmax_ref[pl.ds(j * _CH, _CH)] = past_ref[0]

    @pl.when(i == _PERIODS)
    def _():
        val = jnp.where(shift > 0, latest_ref[pl.ds(j * _CH, _CH)], past_ref[0])
        upd_ref[0] = val
        dzero_ref[pl.ds(j * _CH, _CH)] = val


@functools.partial(jax.jit, static_argnames=())
def _run(past, latest, shift):
    f0, f1, f2 = past.shape[1:]
    jgrid = f0 // _CH
    grid = (_SLOTS, jgrid)
    field = jax.ShapeDtypeStruct((f0, f1, f2), past.dtype)
    return pl.pallas_call(
        _body,
        grid_spec=pltpu.PrefetchScalarGridSpec(
            num_scalar_prefetch=1,
            grid=grid,
            in_specs=[
                pl.BlockSpec(
                    (1, _CH, f1, f2),
                    lambda i, j, s: (jnp.minimum(i + s[0], _SLOTS - 1), j, 0, 0),
                ),
                pl.BlockSpec((f0, f1, f2), lambda i, j, s: (0, 0, 0)),
            ],
            out_specs=[
                pl.BlockSpec((1, _CH, f1, f2), lambda i, j, s: (i, j, 0, 0)),
                pl.BlockSpec((f0, f1, f2), lambda i, j, s: (0, 0, 0)),
                pl.BlockSpec((f0, f1, f2), lambda i, j, s: (0, 0, 0)),
            ],
        ),
        out_shape=[
            jax.ShapeDtypeStruct(past.shape, past.dtype),
            field,
            field,
        ],
    )(shift, past, latest)


def kernel(past, latest, dt_mod_freq, timedelta_seconds):
    dt = dt_mod_freq[0] + jnp.float32(timedelta_seconds)
    is_update_step = dt >= _RESOLUTION_S
    new_dt = jnp.where(is_update_step, dt - _RESOLUTION_S, dt)
    shift = is_update_step.astype(jnp.int32).reshape((1,))
    updated_past, diag_max, diag_zero = _run(past, latest, shift)
    return updated_past, diag_max, diag_zero, new_dt
